# baseline (speedup 1.0000x reference)
import functools

import jax
import jax.numpy as jnp
from jax import lax
from jax.experimental import pallas as pl
from jax.experimental.pallas import tpu as pltpu
from jax.experimental.pallas import tpu_sc as plsc

N_NODES = 10000
N_EDGES = 320000
D_FEAT = 128
NHID = 128
NCLASS = 16

NC = 2
NS = 16
DH = NHID // 2
CHUNK = 128
CHUNK2 = 256
CH1 = 160
CH2 = 40
E_PAD = CHUNK * CH1 * NS
NROWS = 10240
RPS = NROWS // NS
ZR = 64

_SC_PARAMS = pltpu.CompilerParams(use_tc_tiling_on_sc=False)
_MESH = plsc.VectorSubcoreMesh(core_axis_name="c", subcore_axis_name="s")


def _zero_acc(zeros_v, acc, sid, d, sem):
  def zrow(r, _):
    for t in range(d // 16):
      zeros_v[r, pl.ds(t * 16, 16)] = jnp.zeros((16,), jnp.float32)
    return 0
  lax.fori_loop(0, ZR, zrow, 0)
  base = sid * RPS
  for i in range(RPS // ZR):
    pltpu.async_copy(zeros_v, acc.at[pl.ds(base + i * ZR, ZR)], sem)
  for i in range(RPS // ZR):
    pltpu.make_async_copy(zeros_v, acc.at[pl.ds(base, ZR)], sem).wait()
  return base


NBUF1 = 5
NBUF2 = 8


def _edge_pipeline(n_chunks, nbuf, table, src_v, dst_v, rows_v, acc,
                   gsems, ssems):
  for k in range(nbuf - 2):
    pltpu.async_copy(table.at[src_v.at[k]], rows_v.at[k], gsems[k])

  unroll = nbuf if nbuf % 2 == 0 else 2 * nbuf

  def body(j, _):
    for b2 in range(unroll):
      @pl.when(j % unroll == b2)
      def _():
        b = b2 % nbuf
        bn = (b + nbuf - 2) % nbuf
        sp = b2 % 2

        @pl.when(j >= 2)
        def _():
          pltpu.make_async_copy(
              rows_v.at[bn], acc.at[dst_v.at[j - 2]], ssems[sp]).wait()

        @pl.when(j + nbuf - 2 < n_chunks)
        def _():
          pltpu.async_copy(table.at[src_v.at[j + nbuf - 2]], rows_v.at[bn],
                           gsems[bn])

        pltpu.make_async_copy(table.at[src_v.at[j]], rows_v.at[b],
                              gsems[b]).wait()
        pltpu.async_copy(rows_v.at[b], acc.at[dst_v.at[j]], ssems[sp],
                         add=True)
    return 0

  lax.fori_loop(0, n_chunks, body, 0)
  pltpu.make_async_copy(rows_v.at[(n_chunks - 2) % nbuf],
                        acc.at[dst_v.at[n_chunks - 2]],
                        ssems[n_chunks % 2]).wait()
  pltpu.make_async_copy(rows_v.at[(n_chunks - 1) % nbuf],
                        acc.at[dst_v.at[n_chunks - 1]],
                        ssems[(n_chunks + 1) % 2]).wait()


@functools.partial(
    pl.kernel,
    out_type=jax.ShapeDtypeStruct((NC, NROWS, DH), jnp.float32),
    mesh=_MESH,
    compiler_params=_SC_PARAMS,
    scratch_types=[
        pltpu.VMEM((CH1, CHUNK), jnp.int32),
        pltpu.VMEM((CH1, CHUNK), jnp.int32),
        pltpu.VMEM((NBUF1, CHUNK, DH), jnp.float32),
        pltpu.VMEM((ZR, DH), jnp.float32),
        pltpu.VMEM_SHARED((NROWS, DH), jnp.float32),
    ] + [pltpu.SemaphoreType.DMA] * (NBUF1 + 2),
)
def _agg_l1(h_hbm, src_hbm, dst_hbm, out_hbm,
            src_v, dst_v, rows_v, zeros_v, acc,
            *sems):
  cid = lax.axis_index("c")
  sid = lax.axis_index("s")
  pltpu.sync_copy(src_hbm.at[sid], src_v)
  pltpu.sync_copy(dst_hbm.at[sid], dst_v)

  def fix(j, _):
    for t in range(CHUNK // 16):
      sl = pl.ds(t * 16, 16)
      src_v[j, sl] = src_v[j, sl] * 2 + cid
    return 0
  lax.fori_loop(0, CH1, fix, 0)

  base = _zero_acc(zeros_v, acc, sid, DH, sems[NBUF1])
  plsc.subcore_barrier()
  _edge_pipeline(CH1, NBUF1, h_hbm, src_v, dst_v, rows_v, acc,
                 sems[:NBUF1], sems[NBUF1:])
  plsc.subcore_barrier()
  pltpu.sync_copy(acc.at[pl.ds(base, RPS)], out_hbm.at[cid, pl.ds(base, RPS)])


@functools.partial(
    pl.kernel,
    out_type=jax.ShapeDtypeStruct((NC, NROWS, NCLASS), jnp.float32),
    mesh=_MESH,
    compiler_params=_SC_PARAMS,
    scratch_types=[
        pltpu.VMEM((CH2, CHUNK2), jnp.int32),
        pltpu.VMEM((CH2, CHUNK2), jnp.int32),
        pltpu.VMEM((NBUF2, CHUNK2, NCLASS), jnp.float32),
        pltpu.VMEM((ZR, NCLASS), jnp.float32),
        pltpu.VMEM_SHARED((NROWS, NCLASS), jnp.float32),
    ] + [pltpu.SemaphoreType.DMA] * (NBUF2 + 2),
)
def _agg_l2(h_hbm, src_hbm, dst_hbm, out_hbm,
            src_v, dst_v, rows_v, zeros_v, acc,
            *sems):
  cid = lax.axis_index("c")
  sid = lax.axis_index("s")
  pltpu.sync_copy(src_hbm.at[cid, sid], src_v)
  pltpu.sync_copy(dst_hbm.at[cid, sid], dst_v)
  base = _zero_acc(zeros_v, acc, sid, NCLASS, sems[NBUF2])
  plsc.subcore_barrier()
  _edge_pipeline(CH2, NBUF2, h_hbm, src_v, dst_v, rows_v, acc,
                 sems[:NBUF2], sems[NBUF2:])
  plsc.subcore_barrier()
  pltpu.sync_copy(acc.at[pl.ds(base, RPS)], out_hbm.at[cid, pl.ds(base, RPS)])


def _mm1(feats, W1):
  def body(x_ref, w_ref, o_ref):
    o_ref[...] = jnp.dot(x_ref[...], w_ref[...],
                         preferred_element_type=jnp.float32)
  return pl.pallas_call(
      body,
      grid=(5,),
      in_specs=[
          pl.BlockSpec((2000, D_FEAT), lambda i: (i, 0)),
          pl.BlockSpec((D_FEAT, NHID), lambda i: (0, 0)),
      ],
      out_specs=pl.BlockSpec((2000, NHID), lambda i: (i, 0)),
      out_shape=jax.ShapeDtypeStruct((N_NODES, NHID), jnp.float32),
  )(feats, W1)


def _layer2_in(p1pk, b1pk, W2bd):
  def body(pa_ref, pb_ref, ba_ref, bb_ref, wa_ref, wb_ref, o_ref):
    xa = jnp.maximum(pa_ref[0] + ba_ref[0], 0.0)
    xb = jnp.maximum(pb_ref[0] + bb_ref[0], 0.0)
    o_ref[...] = (jnp.dot(xa, wa_ref[0], preferred_element_type=jnp.float32)
                  + jnp.dot(xb, wb_ref[0], preferred_element_type=jnp.float32))
  return pl.pallas_call(
      body,
      grid=(5,),
      in_specs=[
          pl.BlockSpec((1, 1024, 128), lambda i: (0, i, 0)),
          pl.BlockSpec((1, 1024, 128), lambda i: (1, i, 0)),
          pl.BlockSpec((1, 1, 128), lambda i: (0, 0, 0)),
          pl.BlockSpec((1, 1, 128), lambda i: (1, 0, 0)),
          pl.BlockSpec((1, 128, 2 * NCLASS), lambda i: (0, 0, 0)),
          pl.BlockSpec((1, 128, 2 * NCLASS), lambda i: (1, 0, 0)),
      ],
      out_specs=pl.BlockSpec((1024, 2 * NCLASS), lambda i: (i, 0)),
      out_shape=jax.ShapeDtypeStruct((NROWS // 2, 2 * NCLASS), jnp.float32),
  )(p1pk, p1pk, b1pk, b1pk, W2bd, W2bd)


def _finish(p2pk, b2pk):
  NEG = -1e30

  def body(p_ref, b_ref, o_ref):
    x = p_ref[0] + p_ref[1] + b_ref[...]
    lpos = lax.broadcasted_iota(jnp.int32, x.shape, 1) % 16
    m = x
    for k in (1, 2, 4, 8):
      sh = pltpu.roll(m, 128 - k, axis=1)
      m = jnp.maximum(m, jnp.where(lpos <= 15 - k, sh, NEG))
    for k in (1, 2, 4, 8):
      sh = pltpu.roll(m, k, axis=1)
      m = jnp.maximum(m, jnp.where(lpos >= k, sh, NEG))
    e = jnp.exp(x - m)
    r = lax.broadcasted_iota(jnp.int32, (128, 128), 0) // 16
    c = lax.broadcasted_iota(jnp.int32, (128, 128), 1) // 16
    bd = (r == c).astype(jnp.float32)
    s = jnp.dot(e, bd, preferred_element_type=jnp.float32)
    o_ref[...] = e / s

  return pl.pallas_call(
      body,
      grid=(1,),
      in_specs=[
          pl.BlockSpec((2, NROWS // 8, 128), lambda i: (0, 0, 0)),
          pl.BlockSpec((1, 128), lambda i: (0, 0)),
      ],
      out_specs=pl.BlockSpec((NROWS // 8, 128), lambda i: (0, 0)),
      out_shape=jax.ShapeDtypeStruct((NROWS // 8, 128), jnp.float32),
  )(p2pk, b2pk)


def kernel(feats, edge_index, W1, b1, W2, b2):
  src = edge_index[0].astype(jnp.int32)
  dst = edge_index[1].astype(jnp.int32)
  pad = E_PAD - N_EDGES
  ar = jnp.arange(pad, dtype=jnp.int32)
  pad_src = ar % N_NODES
  pad_dst = N_NODES + ar % (NROWS - N_NODES)
  src = jnp.concatenate([src, pad_src])
  dst = jnp.concatenate([dst, pad_dst])
  src1 = src.reshape(NS, CH1, CHUNK)
  dst1 = dst.reshape(NS, CH1, CHUNK)
  src2 = src.reshape(NC, NS, CH2, CHUNK2)
  dst2 = dst.reshape(NC, NS, CH2, CHUNK2)

  b1pk = jnp.tile(b1.reshape(NC, 1, DH), (1, 1, 2))
  W2r = W2.reshape(NC, DH, NCLASS)
  zz = jnp.zeros((NC, DH, NCLASS), jnp.float32)
  W2bd = jnp.concatenate([
      jnp.concatenate([W2r, zz], axis=2),
      jnp.concatenate([zz, W2r], axis=2),
  ], axis=1)
  b2pk = jnp.tile(b2, 8).reshape(1, 128)

  h1 = _mm1(feats, W1)
  table1 = h1.reshape(2 * N_NODES, DH)
  p1 = _agg_l1(table1, src1, dst1)
  h2pk = _layer2_in(p1.reshape(NC, NROWS // 2, 128), b1pk, W2bd)
  h2 = h2pk.reshape(NROWS, NCLASS)
  p2 = _agg_l2(h2, src2, dst2)
  out = _finish(p2.reshape(NC, NROWS // 8, 128), b2pk)
  return out.reshape(NROWS, NCLASS)[:N_NODES]


# speedup vs baseline: 1.0054x; 1.0054x over previous
import functools

import jax
import jax.numpy as jnp
from jax import lax
from jax.experimental import pallas as pl
from jax.experimental.pallas import tpu as pltpu
from jax.experimental.pallas import tpu_sc as plsc

N_NODES = 10000
N_EDGES = 320000
D_FEAT = 128
NHID = 128
NCLASS = 16

NC = 2
NS = 16
DH = NHID // 2
CHUNK = 128
CHUNK2 = 128
CH1 = 160
CH2 = 80
E_PAD = CHUNK * CH1 * NS
NROWS = 10240
RPS = NROWS // NS
ZR = 64

_SC_PARAMS = pltpu.CompilerParams(use_tc_tiling_on_sc=False)
_MESH = plsc.VectorSubcoreMesh(core_axis_name="c", subcore_axis_name="s")


def _zero_acc(zeros_v, acc, sid, d, sem):
  def zrow(r, _):
    for t in range(d // 16):
      zeros_v[r, pl.ds(t * 16, 16)] = jnp.zeros((16,), jnp.float32)
    return 0
  lax.fori_loop(0, ZR, zrow, 0)
  base = sid * RPS
  for i in range(RPS // ZR):
    pltpu.async_copy(zeros_v, acc.at[pl.ds(base + i * ZR, ZR)], sem)
  for i in range(RPS // ZR):
    pltpu.make_async_copy(zeros_v, acc.at[pl.ds(base, ZR)], sem).wait()
  return base


NBUF1 = 5
NBUF2 = 8


def _edge_pipeline(n_chunks, nbuf, table, src_v, dst_v, rows_v, acc,
                   gsems, ssems):
  for k in range(nbuf - 2):
    pltpu.async_copy(table.at[src_v.at[k]], rows_v.at[k], gsems[k])

  unroll = nbuf if nbuf % 2 == 0 else 2 * nbuf

  def body(j, _):
    for b2 in range(unroll):
      @pl.when(j % unroll == b2)
      def _():
        b = b2 % nbuf
        bn = (b + nbuf - 2) % nbuf
        sp = b2 % 2

        @pl.when(j >= 2)
        def _():
          pltpu.make_async_copy(
              rows_v.at[bn], acc.at[dst_v.at[j - 2]], ssems[sp]).wait()

        @pl.when(j + nbuf - 2 < n_chunks)
        def _():
          pltpu.async_copy(table.at[src_v.at[j + nbuf - 2]], rows_v.at[bn],
                           gsems[bn])

        pltpu.make_async_copy(table.at[src_v.at[j]], rows_v.at[b],
                              gsems[b]).wait()
        pltpu.async_copy(rows_v.at[b], acc.at[dst_v.at[j]], ssems[sp],
                         add=True)
    return 0

  lax.fori_loop(0, n_chunks, body, 0)
  pltpu.make_async_copy(rows_v.at[(n_chunks - 2) % nbuf],
                        acc.at[dst_v.at[n_chunks - 2]],
                        ssems[n_chunks % 2]).wait()
  pltpu.make_async_copy(rows_v.at[(n_chunks - 1) % nbuf],
                        acc.at[dst_v.at[n_chunks - 1]],
                        ssems[(n_chunks + 1) % 2]).wait()


@functools.partial(
    pl.kernel,
    out_type=jax.ShapeDtypeStruct((NC, NROWS, DH), jnp.float32),
    mesh=_MESH,
    compiler_params=_SC_PARAMS,
    scratch_types=[
        pltpu.VMEM((CH1, CHUNK), jnp.int32),
        pltpu.VMEM((CH1, CHUNK), jnp.int32),
        pltpu.VMEM((NBUF1, CHUNK, DH), jnp.float32),
        pltpu.VMEM((ZR, DH), jnp.float32),
        pltpu.VMEM_SHARED((NROWS, DH), jnp.float32),
    ] + [pltpu.SemaphoreType.DMA] * (NBUF1 + 2),
)
def _agg_l1(h_hbm, src_hbm, dst_hbm, out_hbm,
            src_v, dst_v, rows_v, zeros_v, acc,
            *sems):
  cid = lax.axis_index("c")
  sid = lax.axis_index("s")
  pltpu.sync_copy(src_hbm.at[sid], src_v)
  pltpu.sync_copy(dst_hbm.at[sid], dst_v)

  def fix(j, _):
    for t in range(CHUNK // 16):
      sl = pl.ds(t * 16, 16)
      src_v[j, sl] = src_v[j, sl] * 2 + cid
    return 0
  lax.fori_loop(0, CH1, fix, 0)

  base = _zero_acc(zeros_v, acc, sid, DH, sems[NBUF1])
  plsc.subcore_barrier()
  _edge_pipeline(CH1, NBUF1, h_hbm, src_v, dst_v, rows_v, acc,
                 sems[:NBUF1], sems[NBUF1:])
  plsc.subcore_barrier()
  pltpu.sync_copy(acc.at[pl.ds(base, RPS)], out_hbm.at[cid, pl.ds(base, RPS)])


@functools.partial(
    pl.kernel,
    out_type=jax.ShapeDtypeStruct((NC, NROWS, NCLASS), jnp.float32),
    mesh=_MESH,
    compiler_params=_SC_PARAMS,
    scratch_types=[
        pltpu.VMEM((CH2, CHUNK2), jnp.int32),
        pltpu.VMEM((CH2, CHUNK2), jnp.int32),
        pltpu.VMEM((NBUF2, CHUNK2, NCLASS), jnp.float32),
        pltpu.VMEM((ZR, NCLASS), jnp.float32),
        pltpu.VMEM_SHARED((NROWS, NCLASS), jnp.float32),
    ] + [pltpu.SemaphoreType.DMA] * (NBUF2 + 2),
)
def _agg_l2(h_hbm, src_hbm, dst_hbm, out_hbm,
            src_v, dst_v, rows_v, zeros_v, acc,
            *sems):
  cid = lax.axis_index("c")
  sid = lax.axis_index("s")
  pltpu.sync_copy(src_hbm.at[cid, sid], src_v)
  pltpu.sync_copy(dst_hbm.at[cid, sid], dst_v)
  base = _zero_acc(zeros_v, acc, sid, NCLASS, sems[NBUF2])
  plsc.subcore_barrier()
  _edge_pipeline(CH2, NBUF2, h_hbm, src_v, dst_v, rows_v, acc,
                 sems[:NBUF2], sems[NBUF2:])
  plsc.subcore_barrier()
  pltpu.sync_copy(acc.at[pl.ds(base, RPS)], out_hbm.at[cid, pl.ds(base, RPS)])


def _mm1(feats, W1):
  def body(x_ref, w_ref, o_ref):
    o_ref[...] = jnp.dot(x_ref[...], w_ref[...],
                         preferred_element_type=jnp.float32)
  return pl.pallas_call(
      body,
      grid=(5,),
      in_specs=[
          pl.BlockSpec((2000, D_FEAT), lambda i: (i, 0)),
          pl.BlockSpec((D_FEAT, NHID), lambda i: (0, 0)),
      ],
      out_specs=pl.BlockSpec((2000, NHID), lambda i: (i, 0)),
      out_shape=jax.ShapeDtypeStruct((N_NODES, NHID), jnp.float32),
  )(feats, W1)


def _layer2_in(p1pk, b1pk, W2bd):
  def body(pa_ref, pb_ref, ba_ref, bb_ref, wa_ref, wb_ref, o_ref):
    xa = jnp.maximum(pa_ref[0] + ba_ref[0], 0.0)
    xb = jnp.maximum(pb_ref[0] + bb_ref[0], 0.0)
    o_ref[...] = (jnp.dot(xa, wa_ref[0], preferred_element_type=jnp.float32)
                  + jnp.dot(xb, wb_ref[0], preferred_element_type=jnp.float32))
  return pl.pallas_call(
      body,
      grid=(5,),
      in_specs=[
          pl.BlockSpec((1, 1024, 128), lambda i: (0, i, 0)),
          pl.BlockSpec((1, 1024, 128), lambda i: (1, i, 0)),
          pl.BlockSpec((1, 1, 128), lambda i: (0, 0, 0)),
          pl.BlockSpec((1, 1, 128), lambda i: (1, 0, 0)),
          pl.BlockSpec((1, 128, 2 * NCLASS), lambda i: (0, 0, 0)),
          pl.BlockSpec((1, 128, 2 * NCLASS), lambda i: (1, 0, 0)),
      ],
      out_specs=pl.BlockSpec((1024, 2 * NCLASS), lambda i: (i, 0)),
      out_shape=jax.ShapeDtypeStruct((NROWS // 2, 2 * NCLASS), jnp.float32),
  )(p1pk, p1pk, b1pk, b1pk, W2bd, W2bd)


def _finish(p2pk, b2pk):
  NEG = -1e30

  def body(p_ref, b_ref, o_ref):
    x = p_ref[0] + p_ref[1] + b_ref[...]
    lpos = lax.broadcasted_iota(jnp.int32, x.shape, 1) % 16
    m = x
    for k in (1, 2, 4, 8):
      sh = pltpu.roll(m, 128 - k, axis=1)
      m = jnp.maximum(m, jnp.where(lpos <= 15 - k, sh, NEG))
    for k in (1, 2, 4, 8):
      sh = pltpu.roll(m, k, axis=1)
      m = jnp.maximum(m, jnp.where(lpos >= k, sh, NEG))
    e = jnp.exp(x - m)
    r = lax.broadcasted_iota(jnp.int32, (128, 128), 0) // 16
    c = lax.broadcasted_iota(jnp.int32, (128, 128), 1) // 16
    bd = (r == c).astype(jnp.float32)
    s = jnp.dot(e, bd, preferred_element_type=jnp.float32)
    o_ref[...] = e / s

  return pl.pallas_call(
      body,
      grid=(1,),
      in_specs=[
          pl.BlockSpec((2, NROWS // 8, 128), lambda i: (0, 0, 0)),
          pl.BlockSpec((1, 128), lambda i: (0, 0)),
      ],
      out_specs=pl.BlockSpec((NROWS // 8, 128), lambda i: (0, 0)),
      out_shape=jax.ShapeDtypeStruct((NROWS // 8, 128), jnp.float32),
  )(p2pk, b2pk)


def kernel(feats, edge_index, W1, b1, W2, b2):
  src = edge_index[0].astype(jnp.int32)
  dst = edge_index[1].astype(jnp.int32)
  pad = E_PAD - N_EDGES
  ar = jnp.arange(pad, dtype=jnp.int32)
  pad_src = ar % N_NODES
  pad_dst = N_NODES + ar % (NROWS - N_NODES)
  src = jnp.concatenate([src, pad_src])
  dst = jnp.concatenate([dst, pad_dst])
  src1 = src.reshape(NS, CH1, CHUNK)
  dst1 = dst.reshape(NS, CH1, CHUNK)
  src2 = src.reshape(NC, NS, CH2, CHUNK2)
  dst2 = dst.reshape(NC, NS, CH2, CHUNK2)

  b1pk = jnp.tile(b1.reshape(NC, 1, DH), (1, 1, 2))
  W2r = W2.reshape(NC, DH, NCLASS)
  zz = jnp.zeros((NC, DH, NCLASS), jnp.float32)
  W2bd = jnp.concatenate([
      jnp.concatenate([W2r, zz], axis=2),
      jnp.concatenate([zz, W2r], axis=2),
  ], axis=1)
  b2pk = jnp.tile(b2, 8).reshape(1, 128)

  h1 = _mm1(feats, W1)
  table1 = h1.reshape(2 * N_NODES, DH)
  p1 = _agg_l1(table1, src1, dst1)
  h2pk = _layer2_in(p1.reshape(NC, NROWS // 2, 128), b1pk, W2bd)
  h2 = h2pk.reshape(NROWS, NCLASS)
  p2 = _agg_l2(h2, src2, dst2)
  out = _finish(p2.reshape(NC, NROWS // 8, 128), b2pk)
  return out.reshape(NROWS, NCLASS)[:N_NODES]


# speedup vs baseline: 1.0079x; 1.0025x over previous
import functools

import jax
import jax.numpy as jnp
from jax import lax
from jax.experimental import pallas as pl
from jax.experimental.pallas import tpu as pltpu
from jax.experimental.pallas import tpu_sc as plsc

N_NODES = 10000
N_EDGES = 320000
D_FEAT = 128
NHID = 128
NCLASS = 16

NC = 2
NS = 16
DH = NHID // 2
CHUNK = 128
CHUNK2 = 128
CH1 = 160
CH2 = 80
E_PAD = CHUNK * CH1 * NS
NROWS = 10240
RPS = NROWS // NS

_SC_PARAMS = pltpu.CompilerParams(use_tc_tiling_on_sc=False)
_MESH = plsc.VectorSubcoreMesh(core_axis_name="c", subcore_axis_name="s")


def _zero_acc(zeros_v, acc, sid, d, sem):
  def zrow(r, _):
    for t in range(d // 16):
      zeros_v[r, pl.ds(t * 16, 16)] = jnp.zeros((16,), jnp.float32)
    return 0
  lax.fori_loop(0, CHUNK, zrow, 0)
  base = sid * RPS
  for i in range(RPS // CHUNK):
    pltpu.async_copy(zeros_v, acc.at[pl.ds(base + i * CHUNK, CHUNK)], sem)
  for i in range(RPS // CHUNK):
    pltpu.make_async_copy(zeros_v, acc.at[pl.ds(base, CHUNK)], sem).wait()
  return base


NBUF1 = 6
NBUF2 = 8


def _edge_pipeline(n_chunks, nbuf, table, src_v, dst_v, rows_v, acc,
                   gsems, ssems):
  for k in range(nbuf - 2):
    pltpu.async_copy(table.at[src_v.at[k]], rows_v.at[k], gsems[k])

  unroll = nbuf if nbuf % 2 == 0 else 2 * nbuf

  def body(j, _):
    for b2 in range(unroll):
      @pl.when(j % unroll == b2)
      def _():
        b = b2 % nbuf
        bn = (b + nbuf - 2) % nbuf
        sp = b2 % 2

        @pl.when(j >= 2)
        def _():
          pltpu.make_async_copy(
              rows_v.at[bn], acc.at[dst_v.at[j - 2]], ssems[sp]).wait()

        @pl.when(j + nbuf - 2 < n_chunks)
        def _():
          pltpu.async_copy(table.at[src_v.at[j + nbuf - 2]], rows_v.at[bn],
                           gsems[bn])

        pltpu.make_async_copy(table.at[src_v.at[j]], rows_v.at[b],
                              gsems[b]).wait()
        pltpu.async_copy(rows_v.at[b], acc.at[dst_v.at[j]], ssems[sp],
                         add=True)
    return 0

  lax.fori_loop(0, n_chunks, body, 0)
  pltpu.make_async_copy(rows_v.at[(n_chunks - 2) % nbuf],
                        acc.at[dst_v.at[n_chunks - 2]],
                        ssems[n_chunks % 2]).wait()
  pltpu.make_async_copy(rows_v.at[(n_chunks - 1) % nbuf],
                        acc.at[dst_v.at[n_chunks - 1]],
                        ssems[(n_chunks + 1) % 2]).wait()


@functools.partial(
    pl.kernel,
    out_type=jax.ShapeDtypeStruct((NC, NROWS, DH), jnp.float32),
    mesh=_MESH,
    compiler_params=_SC_PARAMS,
    scratch_types=[
        pltpu.VMEM((CH1, CHUNK), jnp.int32),
        pltpu.VMEM((CH1, CHUNK), jnp.int32),
        pltpu.VMEM((NBUF1, CHUNK, DH), jnp.float32),
        pltpu.VMEM_SHARED((NROWS, DH), jnp.float32),
    ] + [pltpu.SemaphoreType.DMA] * (NBUF1 + 2),
)
def _agg_l1(h_hbm, src_hbm, dst_hbm, out_hbm,
            src_v, dst_v, rows_v, acc,
            *sems):
  cid = lax.axis_index("c")
  sid = lax.axis_index("s")
  pltpu.sync_copy(src_hbm.at[sid], src_v)
  pltpu.sync_copy(dst_hbm.at[sid], dst_v)

  def fix(j, _):
    for t in range(CHUNK // 16):
      sl = pl.ds(t * 16, 16)
      src_v[j, sl] = src_v[j, sl] * 2 + cid
    return 0
  lax.fori_loop(0, CH1, fix, 0)

  base = _zero_acc(rows_v.at[0], acc, sid, DH, sems[NBUF1])
  plsc.subcore_barrier()
  _edge_pipeline(CH1, NBUF1, h_hbm, src_v, dst_v, rows_v, acc,
                 sems[:NBUF1], sems[NBUF1:])
  plsc.subcore_barrier()
  pltpu.sync_copy(acc.at[pl.ds(base, RPS)], out_hbm.at[cid, pl.ds(base, RPS)])


@functools.partial(
    pl.kernel,
    out_type=jax.ShapeDtypeStruct((NC, NROWS, NCLASS), jnp.float32),
    mesh=_MESH,
    compiler_params=_SC_PARAMS,
    scratch_types=[
        pltpu.VMEM((CH2, CHUNK2), jnp.int32),
        pltpu.VMEM((CH2, CHUNK2), jnp.int32),
        pltpu.VMEM((NBUF2, CHUNK2, NCLASS), jnp.float32),
        pltpu.VMEM_SHARED((NROWS, NCLASS), jnp.float32),
    ] + [pltpu.SemaphoreType.DMA] * (NBUF2 + 2),
)
def _agg_l2(h_hbm, src_hbm, dst_hbm, out_hbm,
            src_v, dst_v, rows_v, acc,
            *sems):
  cid = lax.axis_index("c")
  sid = lax.axis_index("s")
  pltpu.sync_copy(src_hbm.at[cid, sid], src_v)
  pltpu.sync_copy(dst_hbm.at[cid, sid], dst_v)
  base = _zero_acc(rows_v.at[0], acc, sid, NCLASS, sems[NBUF2])
  plsc.subcore_barrier()
  _edge_pipeline(CH2, NBUF2, h_hbm, src_v, dst_v, rows_v, acc,
                 sems[:NBUF2], sems[NBUF2:])
  plsc.subcore_barrier()
  pltpu.sync_copy(acc.at[pl.ds(base, RPS)], out_hbm.at[cid, pl.ds(base, RPS)])


def _mm1(feats, W1):
  def body(x_ref, w_ref, o_ref):
    o_ref[...] = jnp.dot(x_ref[...], w_ref[...],
                         preferred_element_type=jnp.float32)
  return pl.pallas_call(
      body,
      grid=(5,),
      in_specs=[
          pl.BlockSpec((2000, D_FEAT), lambda i: (i, 0)),
          pl.BlockSpec((D_FEAT, NHID), lambda i: (0, 0)),
      ],
      out_specs=pl.BlockSpec((2000, NHID), lambda i: (i, 0)),
      out_shape=jax.ShapeDtypeStruct((N_NODES, NHID), jnp.float32),
  )(feats, W1)


def _layer2_in(p1pk, b1pk, W2bd):
  def body(pa_ref, pb_ref, ba_ref, bb_ref, wa_ref, wb_ref, o_ref):
    xa = jnp.maximum(pa_ref[0] + ba_ref[0], 0.0)
    xb = jnp.maximum(pb_ref[0] + bb_ref[0], 0.0)
    o_ref[...] = (jnp.dot(xa, wa_ref[0], preferred_element_type=jnp.float32)
                  + jnp.dot(xb, wb_ref[0], preferred_element_type=jnp.float32))
  return pl.pallas_call(
      body,
      grid=(5,),
      in_specs=[
          pl.BlockSpec((1, 1024, 128), lambda i: (0, i, 0)),
          pl.BlockSpec((1, 1024, 128), lambda i: (1, i, 0)),
          pl.BlockSpec((1, 1, 128), lambda i: (0, 0, 0)),
          pl.BlockSpec((1, 1, 128), lambda i: (1, 0, 0)),
          pl.BlockSpec((1, 128, 2 * NCLASS), lambda i: (0, 0, 0)),
          pl.BlockSpec((1, 128, 2 * NCLASS), lambda i: (1, 0, 0)),
      ],
      out_specs=pl.BlockSpec((1024, 2 * NCLASS), lambda i: (i, 0)),
      out_shape=jax.ShapeDtypeStruct((NROWS // 2, 2 * NCLASS), jnp.float32),
  )(p1pk, p1pk, b1pk, b1pk, W2bd, W2bd)


def _finish(p2pk, b2pk):
  NEG = -1e30

  def body(p_ref, b_ref, o_ref):
    x = p_ref[0] + p_ref[1] + b_ref[...]
    lpos = lax.broadcasted_iota(jnp.int32, x.shape, 1) % 16
    m = x
    for k in (1, 2, 4, 8):
      sh = pltpu.roll(m, 128 - k, axis=1)
      m = jnp.maximum(m, jnp.where(lpos <= 15 - k, sh, NEG))
    for k in (1, 2, 4, 8):
      sh = pltpu.roll(m, k, axis=1)
      m = jnp.maximum(m, jnp.where(lpos >= k, sh, NEG))
    e = jnp.exp(x - m)
    r = lax.broadcasted_iota(jnp.int32, (128, 128), 0) // 16
    c = lax.broadcasted_iota(jnp.int32, (128, 128), 1) // 16
    bd = (r == c).astype(jnp.float32)
    s = jnp.dot(e, bd, preferred_element_type=jnp.float32)
    o_ref[...] = e / s

  return pl.pallas_call(
      body,
      grid=(1,),
      in_specs=[
          pl.BlockSpec((2, NROWS // 8, 128), lambda i: (0, 0, 0)),
          pl.BlockSpec((1, 128), lambda i: (0, 0)),
      ],
      out_specs=pl.BlockSpec((NROWS // 8, 128), lambda i: (0, 0)),
      out_shape=jax.ShapeDtypeStruct((NROWS // 8, 128), jnp.float32),
  )(p2pk, b2pk)


def kernel(feats, edge_index, W1, b1, W2, b2):
  src = edge_index[0].astype(jnp.int32)
  dst = edge_index[1].astype(jnp.int32)
  pad = E_PAD - N_EDGES
  ar = jnp.arange(pad, dtype=jnp.int32)
  pad_src = ar % N_NODES
  pad_dst = N_NODES + ar % (NROWS - N_NODES)
  src = jnp.concatenate([src, pad_src])
  dst = jnp.concatenate([dst, pad_dst])
  src1 = src.reshape(NS, CH1, CHUNK)
  dst1 = dst.reshape(NS, CH1, CHUNK)
  src2 = src.reshape(NC, NS, CH2, CHUNK2)
  dst2 = dst.reshape(NC, NS, CH2, CHUNK2)

  b1pk = jnp.tile(b1.reshape(NC, 1, DH), (1, 1, 2))
  W2r = W2.reshape(NC, DH, NCLASS)
  zz = jnp.zeros((NC, DH, NCLASS), jnp.float32)
  W2bd = jnp.concatenate([
      jnp.concatenate([W2r, zz], axis=2),
      jnp.concatenate([zz, W2r], axis=2),
  ], axis=1)
  b2pk = jnp.tile(b2, 8).reshape(1, 128)

  h1 = _mm1(feats, W1)
  table1 = h1.reshape(2 * N_NODES, DH)
  p1 = _agg_l1(table1, src1, dst1)
  h2pk = _layer2_in(p1.reshape(NC, NROWS // 2, 128), b1pk, W2bd)
  h2 = h2pk.reshape(NROWS, NCLASS)
  p2 = _agg_l2(h2, src2, dst2)
  out = _finish(p2.reshape(NC, NROWS // 8, 128), b2pk)
  return out.reshape(NROWS, NCLASS)[:N_NODES]
